# baseline (speedup 1.0000x reference)
import jax
import jax.numpy as jnp
from jax.experimental import pallas as pl
from jax.experimental.pallas import tpu as pltpu

_B, _T, _D = 16, 576, 256
_K = 512
_DEPTH = 4
_BETA = 0.25
_N = _B * _T
_BN = 2304
_NB = _N // _BN
_NCH = 4


def _rvq_kernel(z_ref, cb_ref, q_ref, dig_ref, idx_ref, counts_ref,
                stats_ref, loss_ref, perp_ref,
                hi_ref, mid_ref, lo_ref, cbn_ref, cb2_ref):
    i = pl.program_id(0)

    @pl.when(i == 0)
    def _init():
        counts_ref[...] = jnp.zeros_like(counts_ref)
        stats_ref[...] = jnp.zeros_like(stats_ref)
        cb = cb_ref[...]
        hi = cb.astype(jnp.bfloat16)
        r1 = cb - hi.astype(jnp.float32)
        mid = r1.astype(jnp.bfloat16)
        lo = (r1 - mid.astype(jnp.float32)).astype(jnp.bfloat16)
        hi_ref[...] = hi.astype(jnp.float32)
        mid_ref[...] = mid.astype(jnp.float32)
        lo_ref[...] = lo.astype(jnp.float32)
        cbn_ref[...] = jnp.sum(cb * cb, axis=2)
        cb2_ref[...] = cb + cb

    _H = _BN // _NCH
    lane_iota = jax.lax.broadcasted_iota(
        jnp.int32, (_H, _K), 1).astype(jnp.float32)
    z = [z_ref[h * _H:(h + 1) * _H, :] for h in range(_NCH)]
    res = list(z)
    indices = [jnp.zeros((_H, 1), jnp.int32) for _ in range(_NCH)]
    oh_acc = [jnp.zeros((_H, _K), jnp.float32) for _ in range(_NCH)]
    dig_cols = [[] for _ in range(_NCH)]
    mult = 1
    for s in range(_DEPTH):
        cbn = cbn_ref[s][None, :]
        for h in range(_NCH):
            r2 = jnp.sum(res[h] * res[h], axis=1, keepdims=True)
            scores2 = jax.lax.dot_general(
                res[h], cb2_ref[s], (((1,), (1,)), ((), ())),
                preferred_element_type=jnp.float32)
            d2 = r2 - scores2 + cbn
            m = jnp.min(d2, axis=1, keepdims=True)
            idxf = jnp.min(jnp.where(d2 == m, lane_iota, float(_K)), axis=1,
                           keepdims=True)
            idx = idxf.astype(jnp.int32)
            oh_b = (lane_iota == idxf).astype(jnp.float32)
            q = (jax.lax.dot_general(
                    oh_b, hi_ref[s], (((1,), (0,)), ((), ())),
                    preferred_element_type=jnp.float32)
                 + jax.lax.dot_general(
                    oh_b, mid_ref[s], (((1,), (0,)), ((), ())),
                    preferred_element_type=jnp.float32)
                 + jax.lax.dot_general(
                    oh_b, lo_ref[s], (((1,), (0,)), ((), ())),
                    preferred_element_type=jnp.float32))
            res[h] = res[h] - q
            dig_cols[h].append(idx)
            indices[h] = indices[h] + idx * mult
            oh_acc[h] = oh_acc[h] + oh_b
        mult = mult * _K

    for h in range(_NCH):
        q_ref[h * _H:(h + 1) * _H, :] = z[h] - res[h]
        dig_ref[h * _H:(h + 1) * _H, :] = jnp.concatenate(dig_cols[h], axis=1)
        idx_ref[h * _H:(h + 1) * _H, :] = indices[h]
    oh_tot = oh_acc[0]
    for h in range(1, _NCH):
        oh_tot = oh_tot + oh_acc[h]
    counts_ref[...] += jnp.sum(oh_tot, axis=0, keepdims=True)
    r2sum = jnp.sum(res[0] * res[0])
    for h in range(1, _NCH):
        r2sum = r2sum + jnp.sum(res[h] * res[h])
    stats_ref[...] += r2sum.reshape(1, 1)

    @pl.when(i == _NB - 1)
    def _fin():
        p = counts_ref[...] * (1.0 / (_N * _DEPTH))
        perp_ref[...] = jnp.exp(-jnp.sum(p * jnp.log(p + 1e-10))).reshape(1, 1)
        mse = stats_ref[...] * (1.0 / (_N * _D))
        loss_ref[...] = (1.0 + _BETA) * mse


def kernel(latents, codebooks, pad_vector):
    del pad_vector
    z = latents.reshape(_N, _D)
    out_shapes = (
        jax.ShapeDtypeStruct((_N, _D), jnp.float32),
        jax.ShapeDtypeStruct((_N, _DEPTH), jnp.int32),
        jax.ShapeDtypeStruct((_N, 1), jnp.int32),
        jax.ShapeDtypeStruct((1, _K), jnp.float32),
        jax.ShapeDtypeStruct((1, 1), jnp.float32),
        jax.ShapeDtypeStruct((1, 1), jnp.float32),
        jax.ShapeDtypeStruct((1, 1), jnp.float32),
    )
    grid = (_NB,)
    q, digs, idxs, _counts, _stats, loss, perp = pl.pallas_call(
        _rvq_kernel,
        grid=grid,
        in_specs=[
            pl.BlockSpec((_BN, _D), lambda i: (i, 0)),
            pl.BlockSpec((_DEPTH, _K, _D), lambda i: (0, 0, 0)),
        ],
        out_specs=(
            pl.BlockSpec((_BN, _D), lambda i: (i, 0)),
            pl.BlockSpec((_BN, _DEPTH), lambda i: (i, 0)),
            pl.BlockSpec((_BN, 1), lambda i: (i, 0)),
            pl.BlockSpec((1, _K), lambda i: (0, 0)),
            pl.BlockSpec((1, 1), lambda i: (0, 0)),
            pl.BlockSpec((1, 1), lambda i: (0, 0)),
            pl.BlockSpec((1, 1), lambda i: (0, 0)),
        ),
        out_shape=out_shapes,
        scratch_shapes=[
            pltpu.VMEM((_DEPTH, _K, _D), jnp.float32),
            pltpu.VMEM((_DEPTH, _K, _D), jnp.float32),
            pltpu.VMEM((_DEPTH, _K, _D), jnp.float32),
            pltpu.VMEM((_DEPTH, _K), jnp.float32),
            pltpu.VMEM((_DEPTH, _K, _D), jnp.float32),
        ],
        compiler_params=pltpu.CompilerParams(
            dimension_semantics=("arbitrary",)),
    )(z, codebooks)

    quantized_st = q.reshape(_B, _T, _D)
    indices = idxs.reshape(_B, _T)
    digits = digs.T.reshape(_DEPTH, _B, _T)
    special_mask = jnp.zeros((_B, _T), dtype=bool)
    return (quantized_st, indices, digits, special_mask,
            loss[0, 0], perp[0, 0])


# speedup vs baseline: 1.0119x; 1.0119x over previous
import jax
import jax.numpy as jnp
from jax.experimental import pallas as pl
from jax.experimental.pallas import tpu as pltpu

_B, _T, _D = 16, 576, 256
_K = 512
_DEPTH = 4
_BETA = 0.25
_N = _B * _T
_BN = 1152
_NB = _N // _BN
_NCH = 2


def _rvq_kernel(z_ref, cb_ref, q_ref, dig_ref, idx_ref, counts_ref,
                stats_ref, loss_ref, perp_ref,
                hi_ref, mid_ref, lo_ref, cbn_ref, cb2_ref):
    i = pl.program_id(0)

    @pl.when(i == 0)
    def _init():
        counts_ref[...] = jnp.zeros_like(counts_ref)
        stats_ref[...] = jnp.zeros_like(stats_ref)
        cb = cb_ref[...]
        hi = cb.astype(jnp.bfloat16)
        r1 = cb - hi.astype(jnp.float32)
        mid = r1.astype(jnp.bfloat16)
        lo = (r1 - mid.astype(jnp.float32)).astype(jnp.bfloat16)
        hi_ref[...] = hi.astype(jnp.float32)
        mid_ref[...] = mid.astype(jnp.float32)
        lo_ref[...] = lo.astype(jnp.float32)
        cbn_ref[...] = jnp.sum(cb * cb, axis=2)
        cb2_ref[...] = cb + cb

    _H = _BN // _NCH
    lane_iota = jax.lax.broadcasted_iota(
        jnp.int32, (_H, _K), 1).astype(jnp.float32)
    z = [z_ref[h * _H:(h + 1) * _H, :] for h in range(_NCH)]
    res = list(z)
    indices = [jnp.zeros((_H, 1), jnp.int32) for _ in range(_NCH)]
    oh_acc = [jnp.zeros((_H, _K), jnp.float32) for _ in range(_NCH)]
    dig_cols = [[] for _ in range(_NCH)]
    mult = 1
    for s in range(_DEPTH):
        cbn = cbn_ref[s][None, :]
        for h in range(_NCH):
            r2 = jnp.sum(res[h] * res[h], axis=1, keepdims=True)
            scores2 = jax.lax.dot_general(
                res[h], cb2_ref[s], (((1,), (1,)), ((), ())),
                preferred_element_type=jnp.float32)
            d2 = r2 - scores2 + cbn
            m = jnp.min(d2, axis=1, keepdims=True)
            idxf = jnp.min(jnp.where(d2 == m, lane_iota, float(_K)), axis=1,
                           keepdims=True)
            idx = idxf.astype(jnp.int32)
            oh_b = (lane_iota == idxf).astype(jnp.float32)
            q = (jax.lax.dot_general(
                    oh_b, hi_ref[s], (((1,), (0,)), ((), ())),
                    preferred_element_type=jnp.float32)
                 + jax.lax.dot_general(
                    oh_b, mid_ref[s], (((1,), (0,)), ((), ())),
                    preferred_element_type=jnp.float32)
                 + jax.lax.dot_general(
                    oh_b, lo_ref[s], (((1,), (0,)), ((), ())),
                    preferred_element_type=jnp.float32))
            res[h] = res[h] - q
            dig_cols[h].append(idx)
            indices[h] = indices[h] + idx * mult
            oh_acc[h] = oh_acc[h] + oh_b
        mult = mult * _K

    for h in range(_NCH):
        q_ref[h * _H:(h + 1) * _H, :] = z[h] - res[h]
        dig_ref[h * _H:(h + 1) * _H, :] = jnp.concatenate(dig_cols[h], axis=1)
        idx_ref[h * _H:(h + 1) * _H, :] = indices[h]
    oh_tot = oh_acc[0]
    for h in range(1, _NCH):
        oh_tot = oh_tot + oh_acc[h]
    counts_ref[...] += jnp.sum(oh_tot, axis=0, keepdims=True)
    r2sum = jnp.sum(res[0] * res[0])
    for h in range(1, _NCH):
        r2sum = r2sum + jnp.sum(res[h] * res[h])
    stats_ref[...] += r2sum.reshape(1, 1)

    @pl.when(i == _NB - 1)
    def _fin():
        p = counts_ref[...] * (1.0 / (_N * _DEPTH))
        perp_ref[...] = jnp.exp(-jnp.sum(p * jnp.log(p + 1e-10))).reshape(1, 1)
        mse = stats_ref[...] * (1.0 / (_N * _D))
        loss_ref[...] = (1.0 + _BETA) * mse


def kernel(latents, codebooks, pad_vector):
    del pad_vector
    z = latents.reshape(_N, _D)
    out_shapes = (
        jax.ShapeDtypeStruct((_N, _D), jnp.float32),
        jax.ShapeDtypeStruct((_N, _DEPTH), jnp.int32),
        jax.ShapeDtypeStruct((_N, 1), jnp.int32),
        jax.ShapeDtypeStruct((1, _K), jnp.float32),
        jax.ShapeDtypeStruct((1, 1), jnp.float32),
        jax.ShapeDtypeStruct((1, 1), jnp.float32),
        jax.ShapeDtypeStruct((1, 1), jnp.float32),
    )
    grid = (_NB,)
    q, digs, idxs, _counts, _stats, loss, perp = pl.pallas_call(
        _rvq_kernel,
        grid=grid,
        in_specs=[
            pl.BlockSpec((_BN, _D), lambda i: (i, 0)),
            pl.BlockSpec((_DEPTH, _K, _D), lambda i: (0, 0, 0)),
        ],
        out_specs=(
            pl.BlockSpec((_BN, _D), lambda i: (i, 0)),
            pl.BlockSpec((_BN, _DEPTH), lambda i: (i, 0)),
            pl.BlockSpec((_BN, 1), lambda i: (i, 0)),
            pl.BlockSpec((1, _K), lambda i: (0, 0)),
            pl.BlockSpec((1, 1), lambda i: (0, 0)),
            pl.BlockSpec((1, 1), lambda i: (0, 0)),
            pl.BlockSpec((1, 1), lambda i: (0, 0)),
        ),
        out_shape=out_shapes,
        scratch_shapes=[
            pltpu.VMEM((_DEPTH, _K, _D), jnp.float32),
            pltpu.VMEM((_DEPTH, _K, _D), jnp.float32),
            pltpu.VMEM((_DEPTH, _K, _D), jnp.float32),
            pltpu.VMEM((_DEPTH, _K), jnp.float32),
            pltpu.VMEM((_DEPTH, _K, _D), jnp.float32),
        ],
        compiler_params=pltpu.CompilerParams(
            dimension_semantics=("arbitrary",)),
    )(z, codebooks)

    quantized_st = q.reshape(_B, _T, _D)
    indices = idxs.reshape(_B, _T)
    digits = digs.T.reshape(_DEPTH, _B, _T)
    special_mask = jnp.zeros((_B, _T), dtype=bool)
    return (quantized_st, indices, digits, special_mask,
            loss[0, 0], perp[0, 0])


# speedup vs baseline: 1.0437x; 1.0315x over previous
import jax
import jax.numpy as jnp
from jax.experimental import pallas as pl
from jax.experimental.pallas import tpu as pltpu

_B, _T, _D = 16, 576, 256
_K = 512
_DEPTH = 4
_BETA = 0.25
_N = _B * _T
_BN = 1536
_NB = _N // _BN
_NCH = 2


def _rvq_kernel(z_ref, cb_ref, q_ref, dig_ref, idx_ref, counts_ref,
                stats_ref, loss_ref, perp_ref,
                hi_ref, mid_ref, lo_ref, cbn_ref, cb2_ref):
    i = pl.program_id(0)

    @pl.when(i == 0)
    def _init():
        counts_ref[...] = jnp.zeros_like(counts_ref)
        stats_ref[...] = jnp.zeros_like(stats_ref)
        cb = cb_ref[...]
        hi = cb.astype(jnp.bfloat16)
        r1 = cb - hi.astype(jnp.float32)
        mid = r1.astype(jnp.bfloat16)
        lo = (r1 - mid.astype(jnp.float32)).astype(jnp.bfloat16)
        hi_ref[...] = hi.astype(jnp.float32)
        mid_ref[...] = mid.astype(jnp.float32)
        lo_ref[...] = lo.astype(jnp.float32)
        cbn_ref[...] = jnp.sum(cb * cb, axis=2)
        cb2_ref[...] = cb + cb

    _H = _BN // _NCH
    lane_iota = jax.lax.broadcasted_iota(
        jnp.int32, (_H, _K), 1).astype(jnp.float32)
    z = [z_ref[h * _H:(h + 1) * _H, :] for h in range(_NCH)]
    res = list(z)
    indices = [jnp.zeros((_H, 1), jnp.int32) for _ in range(_NCH)]
    oh_acc = [jnp.zeros((_H, _K), jnp.float32) for _ in range(_NCH)]
    dig_cols = [[] for _ in range(_NCH)]
    mult = 1
    for s in range(_DEPTH):
        cbn = cbn_ref[s][None, :]
        for h in range(_NCH):
            r2 = jnp.sum(res[h] * res[h], axis=1, keepdims=True)
            scores2 = jax.lax.dot_general(
                res[h], cb2_ref[s], (((1,), (1,)), ((), ())),
                preferred_element_type=jnp.float32)
            d2 = r2 - scores2 + cbn
            m = jnp.min(d2, axis=1, keepdims=True)
            idxf = jnp.min(jnp.where(d2 == m, lane_iota, float(_K)), axis=1,
                           keepdims=True)
            idx = idxf.astype(jnp.int32)
            oh_b = (lane_iota == idxf).astype(jnp.float32)
            q = (jax.lax.dot_general(
                    oh_b, hi_ref[s], (((1,), (0,)), ((), ())),
                    preferred_element_type=jnp.float32)
                 + jax.lax.dot_general(
                    oh_b, mid_ref[s], (((1,), (0,)), ((), ())),
                    preferred_element_type=jnp.float32)
                 + jax.lax.dot_general(
                    oh_b, lo_ref[s], (((1,), (0,)), ((), ())),
                    preferred_element_type=jnp.float32))
            res[h] = res[h] - q
            dig_cols[h].append(idx)
            indices[h] = indices[h] + idx * mult
            oh_acc[h] = oh_acc[h] + oh_b
        mult = mult * _K

    for h in range(_NCH):
        q_ref[h * _H:(h + 1) * _H, :] = z[h] - res[h]
        dig_ref[h * _H:(h + 1) * _H, :] = jnp.concatenate(dig_cols[h], axis=1)
        idx_ref[h * _H:(h + 1) * _H, :] = indices[h]
    oh_tot = oh_acc[0]
    for h in range(1, _NCH):
        oh_tot = oh_tot + oh_acc[h]
    counts_ref[...] += jnp.sum(oh_tot, axis=0, keepdims=True)
    r2sum = jnp.sum(res[0] * res[0])
    for h in range(1, _NCH):
        r2sum = r2sum + jnp.sum(res[h] * res[h])
    stats_ref[...] += r2sum.reshape(1, 1)

    @pl.when(i == _NB - 1)
    def _fin():
        p = counts_ref[...] * (1.0 / (_N * _DEPTH))
        perp_ref[...] = jnp.exp(-jnp.sum(p * jnp.log(p + 1e-10))).reshape(1, 1)
        mse = stats_ref[...] * (1.0 / (_N * _D))
        loss_ref[...] = (1.0 + _BETA) * mse


def kernel(latents, codebooks, pad_vector):
    del pad_vector
    z = latents.reshape(_N, _D)
    out_shapes = (
        jax.ShapeDtypeStruct((_N, _D), jnp.float32),
        jax.ShapeDtypeStruct((_N, _DEPTH), jnp.int32),
        jax.ShapeDtypeStruct((_N, 1), jnp.int32),
        jax.ShapeDtypeStruct((1, _K), jnp.float32),
        jax.ShapeDtypeStruct((1, 1), jnp.float32),
        jax.ShapeDtypeStruct((1, 1), jnp.float32),
        jax.ShapeDtypeStruct((1, 1), jnp.float32),
    )
    grid = (_NB,)
    q, digs, idxs, _counts, _stats, loss, perp = pl.pallas_call(
        _rvq_kernel,
        grid=grid,
        in_specs=[
            pl.BlockSpec((_BN, _D), lambda i: (i, 0)),
            pl.BlockSpec((_DEPTH, _K, _D), lambda i: (0, 0, 0)),
        ],
        out_specs=(
            pl.BlockSpec((_BN, _D), lambda i: (i, 0)),
            pl.BlockSpec((_BN, _DEPTH), lambda i: (i, 0)),
            pl.BlockSpec((_BN, 1), lambda i: (i, 0)),
            pl.BlockSpec((1, _K), lambda i: (0, 0)),
            pl.BlockSpec((1, 1), lambda i: (0, 0)),
            pl.BlockSpec((1, 1), lambda i: (0, 0)),
            pl.BlockSpec((1, 1), lambda i: (0, 0)),
        ),
        out_shape=out_shapes,
        scratch_shapes=[
            pltpu.VMEM((_DEPTH, _K, _D), jnp.float32),
            pltpu.VMEM((_DEPTH, _K, _D), jnp.float32),
            pltpu.VMEM((_DEPTH, _K, _D), jnp.float32),
            pltpu.VMEM((_DEPTH, _K), jnp.float32),
            pltpu.VMEM((_DEPTH, _K, _D), jnp.float32),
        ],
        compiler_params=pltpu.CompilerParams(
            dimension_semantics=("arbitrary",)),
    )(z, codebooks)

    quantized_st = q.reshape(_B, _T, _D)
    indices = idxs.reshape(_B, _T)
    digits = digs.T.reshape(_DEPTH, _B, _T)
    special_mask = jnp.zeros((_B, _T), dtype=bool)
    return (quantized_st, indices, digits, special_mask,
            loss[0, 0], perp[0, 0])
